# baseline (speedup 1.0000x reference)
import functools

import jax
import jax.numpy as jnp
from jax import lax
from jax.experimental import pallas as pl
from jax.experimental.pallas import tpu as pltpu
from jax.experimental.pallas import tpu_sc as plsc

NC = 2
NS = 16
NW = NC * NS
CB = 128
NBUF = 4


def _make_lookup(b_total: int, h_total: int, d: int):
    bw = b_total // NW
    ncb = bw // CB
    assert bw * NW == b_total and ncb * CB == bw and ncb == NBUF
    r_t = d // 8
    tile_w = 8 * CB

    mesh = plsc.VectorSubcoreMesh(core_axis_name="c", subcore_axis_name="s")

    @functools.partial(
        pl.kernel,
        mesh=mesh,
        compiler_params=pltpu.CompilerParams(
            use_tc_tiling_on_sc=False, needs_layout_passes=False),
        out_type=jax.ShapeDtypeStruct(
            (h_total, r_t, b_total // CB, 8, CB), jnp.float32),
        scratch_types=[
            pltpu.VMEM((h_total, bw), jnp.int32),
            pltpu.VMEM((NBUF * CB, d), jnp.float32),
            pltpu.VMEM((NBUF, d, CB), jnp.float32),
            pltpu.SemaphoreType.DMA,
            pltpu.SemaphoreType.DMA,
            pltpu.SemaphoreType.DMA,
            pltpu.SemaphoreType.DMA,
            pltpu.SemaphoreType.DMA,
            pltpu.SemaphoreType.DMA,
            pltpu.SemaphoreType.DMA,
            pltpu.SemaphoreType.DMA,
            pltpu.SemaphoreType.DMA,
        ],
    )
    def lookup_kernel(qt_hbm, table_hbm, out_hbm, idxt_v, rows_v, dst_v,
                      isem, gs0, gs1, gs2, gs3, os0, os1, os2, os3):
        gsems = [gs0, gs1, gs2, gs3]
        osems = [os0, os1, os2, os3]
        wid = lax.axis_index("c") * NS + lax.axis_index("s")
        b0 = wid * bw

        for h in range(h_total):
            pltpu.async_copy(qt_hbm.at[h, pl.ds(b0, bw)], idxt_v.at[h], isem)
        for h in range(h_total):
            pltpu.make_async_copy(
                qt_hbm.at[h, pl.ds(b0, bw)], idxt_v.at[h], isem).wait()

        iota16 = jnp.arange(16, dtype=jnp.int32)

        def gather_start(h, c, b):
            pltpu.async_copy(
                table_hbm.at[idxt_v.at[h, pl.ds(c * CB, CB)]],
                rows_v.at[pl.ds(b * CB, CB)], gsems[b])

        def gather_wait(h, c, b):
            pltpu.make_async_copy(
                table_hbm.at[idxt_v.at[h, pl.ds(c * CB, CB)]],
                rows_v.at[pl.ds(b * CB, CB)], gsems[b]).wait()

        def out_start(h, c, b):
            for r in range(r_t):
                pltpu.async_copy(
                    dst_v.at[b, pl.ds(r * 8, 8), :],
                    out_hbm.at[h, r, wid * NBUF + c],
                    osems[b])

        def out_wait(h, c, b):
            for r in range(r_t):
                pltpu.make_async_copy(
                    dst_v.at[b, pl.ds(r * 8, 8), :],
                    out_hbm.at[h, r, wid * NBUF + c],
                    osems[b]).wait()

        dd_vecs = [iota16 + dd0 * 16 for dd0 in range(d // 16)]

        def transpose(b):
            @plsc.parallel_loop(0, CB, step=1, unroll=16)
            def tbody(j):
                colv = jnp.zeros((16,), jnp.int32) + j
                for dd0 in range(d // 16):
                    v = rows_v[b * CB + j, pl.ds(dd0 * 16, 16)]
                    plsc.store_scatter(dst_v.at[b], [dd_vecs[dd0], colv], v)

        gather_start(0, 0, 0)
        gather_start(0, 1, 1)

        def step(h, c, first_lap, last_k):
            b = c
            if not first_lap:
                out_wait(h - 1, c, b)
            gather_wait(h, c, b)
            transpose(b)
            out_start(h, c, b)
            if not last_k:
                if c < 2:
                    gather_start(h, c + 2, c + 2)
                else:
                    gather_start(h + 1, c - 2, c - 2)

        for c in range(NBUF):
            step(0, c, True, False)

        def body(h, carry):
            for c in range(NBUF):
                step(h, c, False, False)
            return carry

        lax.fori_loop(1, h_total - 1, body, 0)

        hl = h_total - 1
        step(hl, 0, False, False)
        step(hl, 1, False, False)
        step(hl, 2, False, True)
        step(hl, 3, False, True)
        out_wait(hl, 0, 0)
        out_wait(hl, 1, 1)
        out_wait(hl, 2, 2)
        out_wait(hl, 3, 3)

    return lookup_kernel


def kernel(q, table):
    b, h = q.shape
    v, d = table.shape
    qt = q.T.astype(jnp.int32)
    out5 = _make_lookup(b, h, d)(qt, table)
    return out5.transpose(2, 4, 0, 1, 3).reshape(b, h, d)


# speedup vs baseline: 1.6345x; 1.6345x over previous
import functools

import jax
import jax.numpy as jnp
from jax import lax
from jax.experimental import pallas as pl
from jax.experimental.pallas import tpu as pltpu
from jax.experimental.pallas import tpu_sc as plsc

NC = 2
NS = 16
NW = NC * NS
CB = 128
NBUF = 4


def _make_lookup(b_total: int, h_total: int, d: int):
    bw = b_total // NW
    ncb = bw // CB
    assert bw * NW == b_total and ncb * CB == bw and ncb == NBUF
    r_t = d // 8
    tile_w = 8 * CB

    mesh = plsc.VectorSubcoreMesh(core_axis_name="c", subcore_axis_name="s")

    @functools.partial(
        pl.kernel,
        mesh=mesh,
        compiler_params=pltpu.CompilerParams(
            use_tc_tiling_on_sc=False, needs_layout_passes=False),
        out_type=jax.ShapeDtypeStruct(
            (h_total, r_t, b_total // CB, 8, CB), jnp.float32),
        scratch_types=[
            pltpu.VMEM((h_total, bw), jnp.int32),
            pltpu.VMEM((NBUF * CB, d), jnp.float32),
            pltpu.VMEM((NBUF, d, CB + 1), jnp.float32),
            pltpu.SemaphoreType.DMA,
            pltpu.SemaphoreType.DMA,
            pltpu.SemaphoreType.DMA,
            pltpu.SemaphoreType.DMA,
            pltpu.SemaphoreType.DMA,
            pltpu.SemaphoreType.DMA,
            pltpu.SemaphoreType.DMA,
            pltpu.SemaphoreType.DMA,
            pltpu.SemaphoreType.DMA,
        ],
    )
    def lookup_kernel(qt_hbm, table_hbm, out_hbm, idxt_v, rows_v, dst_v,
                      isem, gs0, gs1, gs2, gs3, os0, os1, os2, os3):
        gsems = [gs0, gs1, gs2, gs3]
        osems = [os0, os1, os2, os3]
        wid = lax.axis_index("c") * NS + lax.axis_index("s")
        b0 = wid * bw

        for h in range(h_total):
            pltpu.async_copy(qt_hbm.at[h, pl.ds(b0, bw)], idxt_v.at[h], isem)
        for h in range(h_total):
            pltpu.make_async_copy(
                qt_hbm.at[h, pl.ds(b0, bw)], idxt_v.at[h], isem).wait()

        iota16 = jnp.arange(16, dtype=jnp.int32)

        def gather_start(h, c, b):
            pltpu.async_copy(
                table_hbm.at[idxt_v.at[h, pl.ds(c * CB, CB)]],
                rows_v.at[pl.ds(b * CB, CB)], gsems[b])

        def gather_wait(h, c, b):
            pltpu.make_async_copy(
                table_hbm.at[idxt_v.at[h, pl.ds(c * CB, CB)]],
                rows_v.at[pl.ds(b * CB, CB)], gsems[b]).wait()

        def out_start(h, c, b):
            for r in range(r_t):
                pltpu.async_copy(
                    dst_v.at[b, pl.ds(r * 8, 8), pl.ds(0, CB)],
                    out_hbm.at[h, r, wid * NBUF + c],
                    osems[b])

        def out_wait(h, c, b):
            for r in range(r_t):
                pltpu.make_async_copy(
                    dst_v.at[b, pl.ds(r * 8, 8), pl.ds(0, CB)],
                    out_hbm.at[h, r, wid * NBUF + c],
                    osems[b]).wait()

        dd_vecs = [iota16 + dd0 * 16 for dd0 in range(d // 16)]

        def transpose(b):
            @plsc.parallel_loop(0, CB, step=1, unroll=16)
            def tbody(j):
                colv = jnp.zeros((16,), jnp.int32) + j
                for dd0 in range(d // 16):
                    v = rows_v[b * CB + j, pl.ds(dd0 * 16, 16)]
                    plsc.store_scatter(dst_v.at[b], [dd_vecs[dd0], colv], v)

        gather_start(0, 0, 0)
        gather_start(0, 1, 1)

        def step(h, c, first_lap, last_k):
            b = c
            if not first_lap:
                out_wait(h - 1, c, b)
            gather_wait(h, c, b)
            transpose(b)
            out_start(h, c, b)
            if not last_k:
                if c < 2:
                    gather_start(h, c + 2, c + 2)
                else:
                    gather_start(h + 1, c - 2, c - 2)

        for c in range(NBUF):
            step(0, c, True, False)

        def body(h, carry):
            for c in range(NBUF):
                step(h, c, False, False)
            return carry

        lax.fori_loop(1, h_total - 1, body, 0)

        hl = h_total - 1
        step(hl, 0, False, False)
        step(hl, 1, False, False)
        step(hl, 2, False, True)
        step(hl, 3, False, True)
        out_wait(hl, 0, 0)
        out_wait(hl, 1, 1)
        out_wait(hl, 2, 2)
        out_wait(hl, 3, 3)

    return lookup_kernel


def kernel(q, table):
    b, h = q.shape
    v, d = table.shape
    qt = q.T.astype(jnp.int32)
    out5 = _make_lookup(b, h, d)(qt, table)
    return out5.transpose(2, 4, 0, 1, 3).reshape(b, h, d)


# speedup vs baseline: 1.6348x; 1.0002x over previous
import functools

import jax
import jax.numpy as jnp
from jax import lax
from jax.experimental import pallas as pl
from jax.experimental.pallas import tpu as pltpu
from jax.experimental.pallas import tpu_sc as plsc

NC = 2
NS = 16
NW = NC * NS
CB = 128
NBUF = 4


def _make_lookup(b_total: int, h_total: int, d: int):
    bw = b_total // NW
    ncb = bw // CB
    assert bw * NW == b_total and ncb * CB == bw and ncb == NBUF
    r_t = d // 8

    mesh = plsc.VectorSubcoreMesh(core_axis_name="c", subcore_axis_name="s")

    @functools.partial(
        pl.kernel,
        mesh=mesh,
        compiler_params=pltpu.CompilerParams(
            use_tc_tiling_on_sc=False, needs_layout_passes=False),
        out_type=jax.ShapeDtypeStruct(
            (h_total, r_t, b_total // CB, 8, CB), jnp.float32),
        scratch_types=[
            pltpu.VMEM((h_total, bw), jnp.int32),
            pltpu.VMEM((NBUF * CB, d), jnp.float32),
            pltpu.VMEM((NBUF, d, CB + 1), jnp.float32),
            pltpu.SemaphoreType.DMA,
            pltpu.SemaphoreType.DMA,
            pltpu.SemaphoreType.DMA,
            pltpu.SemaphoreType.DMA,
            pltpu.SemaphoreType.DMA,
            pltpu.SemaphoreType.DMA,
            pltpu.SemaphoreType.DMA,
            pltpu.SemaphoreType.DMA,
            pltpu.SemaphoreType.DMA,
        ],
    )
    def lookup_kernel(qt_hbm, table_hbm, out_hbm, idxt_v, rows_v, dst_v,
                      isem, gs0, gs1, gs2, gs3, os0, os1, os2, os3):
        gsems = [gs0, gs1, gs2, gs3]
        osems = [os0, os1, os2, os3]
        wid = lax.axis_index("c") * NS + lax.axis_index("s")
        b0 = wid * bw

        for h in range(h_total):
            pltpu.async_copy(qt_hbm.at[h, pl.ds(b0, bw)], idxt_v.at[h], isem)
        for h in range(h_total):
            pltpu.make_async_copy(
                qt_hbm.at[h, pl.ds(b0, bw)], idxt_v.at[h], isem).wait()

        iota16 = jnp.arange(16, dtype=jnp.int32)

        def gather_start(h, c, b):
            pltpu.async_copy(
                table_hbm.at[idxt_v.at[h, pl.ds(c * CB, CB)]],
                rows_v.at[pl.ds(b * CB, CB)], gsems[b])

        def gather_wait(h, c, b):
            pltpu.make_async_copy(
                table_hbm.at[idxt_v.at[h, pl.ds(c * CB, CB)]],
                rows_v.at[pl.ds(b * CB, CB)], gsems[b]).wait()

        def out_start(h, c, b):
            for r in range(r_t):
                pltpu.async_copy(
                    dst_v.at[b, pl.ds(r * 8, 8), pl.ds(0, CB)],
                    out_hbm.at[h, r, wid * NBUF + c],
                    osems[b])

        def out_wait(h, c, b):
            for r in range(r_t):
                pltpu.make_async_copy(
                    dst_v.at[b, pl.ds(r * 8, 8), pl.ds(0, CB)],
                    out_hbm.at[h, r, wid * NBUF + c],
                    osems[b]).wait()

        dd_vecs = [iota16 + dd0 * 16 for dd0 in range(d // 16)]

        def transpose(b):
            @plsc.parallel_loop(0, CB, step=1, unroll=16)
            def tbody(j):
                colv = jnp.zeros((16,), jnp.int32) + j
                for dd0 in range(d // 16):
                    v = rows_v[b * CB + j, pl.ds(dd0 * 16, 16)]
                    plsc.store_scatter(dst_v.at[b], [dd_vecs[dd0], colv], v)

        gather_start(0, 0, 0)
        gather_start(0, 1, 1)

        def step(h, c, first_lap, last_k):
            b = c
            if not first_lap:
                out_wait(h - 1, c, b)
            gather_wait(h, c, b)
            transpose(b)
            out_start(h, c, b)
            if not last_k:
                if c < 2:
                    gather_start(h, c + 2, c + 2)
                else:
                    gather_start(h + 1, c - 2, c - 2)

        for c in range(NBUF):
            step(0, c, True, False)

        def body(h, carry):
            for c in range(NBUF):
                step(h, c, False, False)
            return carry

        lax.fori_loop(1, h_total - 1, body, 0)

        hl = h_total - 1
        step(hl, 0, False, False)
        step(hl, 1, False, False)
        step(hl, 2, False, True)
        step(hl, 3, False, True)
        out_wait(hl, 0, 0)
        out_wait(hl, 1, 1)
        out_wait(hl, 2, 2)
        out_wait(hl, 3, 3)

    return lookup_kernel


def kernel(q, table):
    b, h = q.shape
    v, d = table.shape
    qt = q.T.astype(jnp.int32)
    out5 = _make_lookup(b, h, d)(qt, table)
    return out5.transpose(2, 4, 0, 1, 3).reshape(b, h, d)
